# baseline (speedup 1.0000x reference)
import jax
import jax.numpy as jnp
from jax import lax
from jax.experimental import pallas as pl
from jax.experimental.pallas import tpu as pltpu
from jax.experimental.pallas import tpu_sc as plsc

_NC = 2
_NS = 16
_L = 16

_T = 200
_D = 64
_ROW = (_T + 1) * _D
_NW = _NC * _NS
_NBUF = 2
_RPB = 2


def _sc_body(x_hbm, tok_hbm, cls_hbm, pos_hbm, out_hbm,
             posv, xv, outb0, outb1, sem0, sem1):
    spw = out_hbm.shape[0] // _NW
    xpw = spw * _RPB * _T
    wid = lax.axis_index("s") * _NC + lax.axis_index("c")
    rb = wid * spw
    outb = (outb0, outb1)
    sems = (sem0, sem1)

    pltpu.sync_copy(pos_hbm, posv.at[pl.ds(0, _ROW)])
    pltpu.sync_copy(tok_hbm, posv.at[pl.ds(_ROW, 2 * _D)])
    pltpu.sync_copy(cls_hbm, posv.at[pl.ds(_ROW + 2 * _D, _D)])
    pltpu.sync_copy(x_hbm.at[pl.ds(wid * xpw, xpw)], xv.at[pl.ds(0, xpw)])

    e0 = [posv[pl.ds(_ROW + c * _L, _L)] for c in range(4)]
    e1 = [posv[pl.ds(_ROW + _D + c * _L, _L)] for c in range(4)]
    clsv = [posv[pl.ds(_ROW + 2 * _D + c * _L, _L)] for c in range(4)]
    diffs = [e1[c] - e0[c] for c in range(4)]

    def fold(t, carry):
        for c in range(4):
            o = t * _D + c * _L
            posv[pl.ds(o, _L)] = posv[pl.ds(o, _L)] + e0[c]
        return carry
    lax.fori_loop(0, _T, fold, 0)
    for c in range(4):
        o = _T * _D + c * _L
        posv[pl.ds(o, _L)] = posv[pl.ds(o, _L)] + clsv[c]

    def emit_token(ob, xf, j, ob_off, pos_off):
        xb = lax.gather(
            xf, jnp.full((_L, 1), j, jnp.int32),
            lax.GatherDimensionNumbers(offset_dims=(), collapsed_slice_dims=(0,),
                                       start_index_map=(0,)),
            (1,), mode=lax.GatherScatterMode.PROMISE_IN_BOUNDS)
        for c in range(4):
            ob[pl.ds(ob_off + c * _L, _L)] = (
                posv[pl.ds(pos_off + c * _L, _L)] + xb * diffs[c])

    def compute_rows(ls, ob):
        for r in range(_RPB):
            xrow = (ls * _RPB + r) * _T
            ob_base = r * _ROW

            def chunk(tc, carry, xrow=xrow, ob_base=ob_base):
                xf = xv[pl.ds(xrow + tc * _L, _L)].astype(jnp.float32)
                for j in range(_L):
                    tb = (tc * _L + j) * _D
                    emit_token(ob, xf, j, ob_base + tb, tb)
                return carry
            lax.fori_loop(0, _T // _L, chunk, 0)
            nfull = (_T // _L) * _L
            xf = xv[pl.ds(xrow + nfull, _L)].astype(jnp.float32)
            for j in range(_T - nfull):
                tb = (nfull + j) * _D
                emit_token(ob, xf, j, ob_base + tb, tb)
            for c in range(4):
                ob[pl.ds(ob_base + _T * _D + c * _L, _L)] = (
                    posv[pl.ds(_T * _D + c * _L, _L)])

    def group(g, carry):
        for k in range(_NBUF):
            ls = g * _NBUF + k

            @pl.when(g > 0)
            def _wait():
                pltpu.make_async_copy(outb[k], out_hbm.at[rb + ls],
                                      sems[k]).wait()

            compute_rows(ls, outb[k])
            pltpu.async_copy(outb[k], out_hbm.at[rb + ls], sems[k])
        return carry

    lax.fori_loop(0, spw // _NBUF, group, 0)
    for k in range(_NBUF):
        pltpu.make_async_copy(outb[k], out_hbm.at[rb], sems[k]).wait()


def kernel(x, token_embedding, cls, position_embedding):
    B, T = x.shape
    D = token_embedding.shape[1]
    xpw = B * _T // _NW
    mesh = plsc.VectorSubcoreMesh(core_axis_name="c", subcore_axis_name="s")
    run = pl.kernel(
        _sc_body,
        mesh=mesh,
        out_type=jax.ShapeDtypeStruct((B // _RPB, _RPB * _ROW), jnp.float32),
        scratch_types=[
            pltpu.VMEM((_ROW + 3 * _D,), jnp.float32),
            pltpu.VMEM((xpw + _L,), jnp.int32),
            pltpu.VMEM((_RPB * _ROW,), jnp.float32),
            pltpu.VMEM((_RPB * _ROW,), jnp.float32),
            pltpu.SemaphoreType.DMA,
            pltpu.SemaphoreType.DMA,
        ],
    )
    out = run(
        x.astype(jnp.int32).reshape(-1),
        token_embedding.reshape(-1),
        cls.reshape(-1),
        position_embedding.reshape(-1),
    )
    return out.reshape(B, T + 1, D)


# speedup vs baseline: 2.1976x; 2.1976x over previous
import jax
import jax.numpy as jnp
from jax import lax
from jax.experimental import pallas as pl
from jax.experimental.pallas import tpu as pltpu
from jax.experimental.pallas import tpu_sc as plsc

_NC = 2
_NS = 16
_L = 16

_T = 200
_D = 64
_ROW = (_T + 1) * _D
_NW = _NC * _NS
_NBUF = 2
_RPB = 1


def _sc_body(x_hbm, tok_hbm, cls_hbm, pos_hbm, out_hbm,
             posv, xv, outb0, outb1, sem0, sem1):
    spw = out_hbm.shape[0] // _NW
    xpw = spw * _RPB * _T
    wid = lax.axis_index("s") * _NC + lax.axis_index("c")
    rb = wid * spw
    outb = (outb0, outb1)
    sems = (sem0, sem1)

    pltpu.sync_copy(pos_hbm, posv.at[pl.ds(0, _ROW)])
    pltpu.sync_copy(tok_hbm, posv.at[pl.ds(_ROW, 2 * _D)])
    pltpu.sync_copy(cls_hbm, posv.at[pl.ds(_ROW + 2 * _D, _D)])
    pltpu.sync_copy(x_hbm.at[pl.ds(wid * xpw, xpw)], xv.at[pl.ds(0, xpw)])

    e0 = [posv[pl.ds(_ROW + c * _L, _L)] for c in range(4)]
    e1 = [posv[pl.ds(_ROW + _D + c * _L, _L)] for c in range(4)]
    clsv = [posv[pl.ds(_ROW + 2 * _D + c * _L, _L)] for c in range(4)]
    diffs = [e1[c] - e0[c] for c in range(4)]

    def fold(t, carry):
        for c in range(4):
            o = t * _D + c * _L
            posv[pl.ds(o, _L)] = posv[pl.ds(o, _L)] + e0[c]
        return carry
    lax.fori_loop(0, _T, fold, 0)
    for c in range(4):
        o = _T * _D + c * _L
        posv[pl.ds(o, _L)] = posv[pl.ds(o, _L)] + clsv[c]

    def emit_token(ob, xf, j, ob_off, pos_off):
        xb = lax.gather(
            xf, jnp.full((_L, 1), j, jnp.int32),
            lax.GatherDimensionNumbers(offset_dims=(), collapsed_slice_dims=(0,),
                                       start_index_map=(0,)),
            (1,), mode=lax.GatherScatterMode.PROMISE_IN_BOUNDS)
        for c in range(4):
            ob[pl.ds(ob_off + c * _L, _L)] = (
                posv[pl.ds(pos_off + c * _L, _L)] + xb * diffs[c])

    def compute_rows(ls, ob):
        for r in range(_RPB):
            xrow = (ls * _RPB + r) * _T
            ob_base = r * _ROW

            def chunk(tc, carry, xrow=xrow, ob_base=ob_base):
                xf = xv[pl.ds(xrow + tc * _L, _L)].astype(jnp.float32)
                for j in range(_L):
                    tb = (tc * _L + j) * _D
                    emit_token(ob, xf, j, ob_base + tb, tb)
                return carry
            lax.fori_loop(0, _T // _L, chunk, 0)
            nfull = (_T // _L) * _L
            xf = xv[pl.ds(xrow + nfull, _L)].astype(jnp.float32)
            for j in range(_T - nfull):
                tb = (nfull + j) * _D
                emit_token(ob, xf, j, ob_base + tb, tb)
            for c in range(4):
                ob[pl.ds(ob_base + _T * _D + c * _L, _L)] = (
                    posv[pl.ds(_T * _D + c * _L, _L)])

    def group(g, carry):
        for k in range(_NBUF):
            ls = g * _NBUF + k

            @pl.when(g > 0)
            def _wait():
                pltpu.make_async_copy(outb[k], out_hbm.at[rb + ls],
                                      sems[k]).wait()

            compute_rows(ls, outb[k])
            pltpu.async_copy(outb[k], out_hbm.at[rb + ls], sems[k])
        return carry

    lax.fori_loop(0, spw // _NBUF, group, 0)
    for k in range(_NBUF):
        pltpu.make_async_copy(outb[k], out_hbm.at[rb], sems[k]).wait()


def kernel(x, token_embedding, cls, position_embedding):
    B, T = x.shape
    D = token_embedding.shape[1]
    xpw = B * _T // _NW
    mesh = plsc.VectorSubcoreMesh(core_axis_name="c", subcore_axis_name="s")
    run = pl.kernel(
        _sc_body,
        mesh=mesh,
        out_type=jax.ShapeDtypeStruct((B // _RPB, _RPB * _ROW), jnp.float32),
        scratch_types=[
            pltpu.VMEM((_ROW + 3 * _D,), jnp.float32),
            pltpu.VMEM((xpw + _L,), jnp.int32),
            pltpu.VMEM((_RPB * _ROW,), jnp.float32),
            pltpu.VMEM((_RPB * _ROW,), jnp.float32),
            pltpu.SemaphoreType.DMA,
            pltpu.SemaphoreType.DMA,
        ],
    )
    out = run(
        x.astype(jnp.int32).reshape(-1),
        token_embedding.reshape(-1),
        cls.reshape(-1),
        position_embedding.reshape(-1),
    )
    return out.reshape(B, T + 1, D)
